# baseline (speedup 1.0000x reference)
import jax, jax.numpy as jnp
from jax.experimental import pallas as pl


def kernel(ent_name, ent_embed_in, edge_index, edge_type_r, r_head, r_head_sum, r_tail, r_tail_sum, kW, kb, qW, qb, vW, vb, outW, outb, attr_r, relL, relR, rel_skip_w):
    src = edge_index[0]; dst = edge_index[1]
    n = ent_name.shape[0]
    H, DK = kW.shape[0], kW.shape[2]
    inv_sqrt = 1.0 / jnp.sqrt(jnp.float32(DK))
    k_all = jnp.einsum('nd,hdk->nhk', ent_name, kW) + kb
    q_all = jnp.einsum('nd,hdk->nhk', ent_name, qW) + qb
    v_all = jnp.einsum('nd,hdk->nhk', ent_name, vW) + vb
    P_L = jnp.einsum('nd,hdk->nhk', ent_name, relL)
    P_R = jnp.einsum('nd,hdk->nhk', ent_name, relR)
    L_r = jnp.einsum('rn,nhk->rhk', r_head, P_L) * r_head_sum[:, :, None]
    R_r = jnp.einsum('rn,nhk->rhk', r_tail, P_R) * r_tail_sum[:, :, None]
    rel = jax.nn.relu(jnp.concatenate([L_r, R_r], axis=-1))
    relA = rel[..., :DK] * attr_r[None, :, :DK, 0]
    relB = rel[..., DK:] * attr_r[None, :, DK:, 0]
    S = jnp.einsum('nhk,rhk->nrh', k_all, relA)
    T = jnp.einsum('nhk,rhk->nrh', q_all, relB)
    att = (S[src, edge_type_r] + T[dst, edge_type_r]) * inv_sqrt
    att = -jax.nn.leaky_relu(att, negative_slope=0.2)
    C = jnp.max(att, axis=0)
    e = jnp.exp(att - C[None, :])
    A_v = jax.ops.segment_sum(e[:, :, None] * v_all[src], dst, num_segments=n)
    A_r = jax.ops.segment_sum(e[:, :, None] * rel[edge_type_r], dst, num_segments=n)
    denom = jax.ops.segment_sum(e, dst, num_segments=n)
    inv = 1.0 / (denom + 1e-16)
    zv = jax.nn.relu(A_v * inv[:, :, None])
    zr = jax.nn.relu(A_r * inv[:, :, None])
    out = (jnp.einsum('nhk,hko->nho', zv, outW[:, :DK, :]) +
           jnp.einsum('nhc,hco->nho', zr, outW[:, DK:, :]) + outb)
    out = out.reshape(n, H * DK)
    a = jax.nn.sigmoid(rel_skip_w[0])
    return out * a + ent_embed_in * (1.0 - a)


# speedup vs baseline: 5.3778x; 5.3778x over previous
import functools

import jax
import jax.numpy as jnp
from jax import lax
from jax.experimental import pallas as pl
from jax.experimental.pallas import tpu as pltpu
from jax.experimental.pallas import tpu_sc as plsc

N = 10000
E = 160000
R = 200
H = 4
DK = 32
ALPHA = 0.2
INP = 384
KP = 10112
RP = 224
NC = 2
NS = 16
NW = NC * NS
CB = 128
NBLK = E // CB
_f32 = jnp.float32
_i32 = jnp.int32


def _nblk_of(wid):
    return jnp.where(wid < NBLK % NW, NBLK // NW + 1, NBLK // NW)


def _proj_mm(ent_pad, wcat, bcat8):
    def body(x_ref, w_ref, b_ref, o_ref):
        o_ref[...] = jnp.dot(x_ref[...], w_ref[...],
                             preferred_element_type=_f32) + b_ref[...][0:1, :]
    return pl.pallas_call(
        body,
        grid=(10,),
        in_specs=[
            pl.BlockSpec((N // 10, INP), lambda i: (i, 0)),
            pl.BlockSpec((INP, 640), lambda i: (0, 0)),
            pl.BlockSpec((8, 640), lambda i: (0, 0)),
        ],
        out_specs=pl.BlockSpec((N // 10, 640), lambda i: (i, 0)),
        out_shape=jax.ShapeDtypeStruct((N, 640), _f32),
    )(ent_pad, wcat, bcat8)


def _rel_mm(r_head_p, r_tail_p, pmat_p, hs128, ts128):
    steps = KP // 128

    def body(rh_ref, rt_ref, p_ref, hs_ref, ts_ref, o_ref, acc_ref):
        k = pl.program_id(0)

        @pl.when(k == 0)
        def _():
            acc_ref[...] = jnp.zeros_like(acc_ref)

        p = p_ref[...]
        acc_ref[:, :128] += jnp.dot(rh_ref[...], p[:, :128],
                                    preferred_element_type=_f32)
        acc_ref[:, 128:] += jnp.dot(rt_ref[...], p[:, 128:],
                                    preferred_element_type=_f32)

        @pl.when(k == steps - 1)
        def _():
            scale = jnp.concatenate([hs_ref[...], ts_ref[...]], axis=1)
            o_ref[...] = jnp.maximum(acc_ref[...] * scale, 0.0)

    return pl.pallas_call(
        body,
        grid=(steps,),
        in_specs=[
            pl.BlockSpec((R, 128), lambda k: (0, k)),
            pl.BlockSpec((R, 128), lambda k: (0, k)),
            pl.BlockSpec((128, 256), lambda k: (k, 0)),
            pl.BlockSpec((R, 128), lambda k: (0, 0)),
            pl.BlockSpec((R, 128), lambda k: (0, 0)),
        ],
        out_specs=pl.BlockSpec((R, 256), lambda k: (0, 0)),
        out_shape=jax.ShapeDtypeStruct((R, 256), _f32),
        scratch_shapes=[pltpu.VMEM((R, 256), _f32)],
    )(r_head_p, r_tail_p, pmat_p, hs128, ts128)


def _st_mm(kq, ms, mt):
    def body(kq_ref, ms_ref, mt_ref, s_ref, t_ref):
        kqv = kq_ref[...]
        s_ref[...] = jnp.dot(kqv[:, :128], ms_ref[...],
                             preferred_element_type=_f32)
        t_ref[...] = jnp.dot(kqv[:, 128:], mt_ref[...],
                             preferred_element_type=_f32)
    return pl.pallas_call(
        body,
        grid=(10,),
        in_specs=[
            pl.BlockSpec((N // 10, 256), lambda i: (i, 0)),
            pl.BlockSpec((128, RP * H), lambda i: (0, 0)),
            pl.BlockSpec((128, RP * H), lambda i: (0, 0)),
        ],
        out_specs=[
            pl.BlockSpec((N // 10, RP * H), lambda i: (i, 0)),
            pl.BlockSpec((N // 10, RP * H), lambda i: (i, 0)),
        ],
        out_shape=[jax.ShapeDtypeStruct((N, RP * H), _f32),
                   jax.ShapeDtypeStruct((N, RP * H), _f32)],
    )(kq, ms, mt)


def _eidx_mm(src2, dst2, et2):
    def body(s_ref, d_ref, t_ref, si_ref, di_ref):
        t = t_ref[...]
        si_ref[...] = s_ref[...] * RP + t
        di_ref[...] = d_ref[...] * RP + t
    rows = E // 128
    return pl.pallas_call(
        body,
        grid=(1,),
        in_specs=[pl.BlockSpec((rows, 128), lambda i: (0, 0))] * 3,
        out_specs=[pl.BlockSpec((rows, 128), lambda i: (0, 0))] * 2,
        out_shape=[jax.ShapeDtypeStruct((rows, 128), _i32)] * 2,
    )(src2, dst2, et2)


def _final_mm(ar0, ar1, den0, den1, av0, av1, ent, ov, orr, outb8, exp4, exp8,
              a2):
    def body(r0_ref, r1_ref, d0_ref, d1_ref, a0_ref, a1_ref, e_ref, ov_ref,
             or_ref, ob_ref, e4_ref, e8_ref, sa_ref, o_ref):
        denom = d0_ref[...][:, 0:4] + d1_ref[...][:, 0:4]
        inv = 1.0 / (denom + 1e-16)
        inv128 = jnp.dot(inv, e4_ref[...][0:4, :], preferred_element_type=_f32)
        inv256 = jnp.dot(inv, e8_ref[...][0:4, :], preferred_element_type=_f32)
        av = a0_ref[...] + a1_ref[...]
        zv = jnp.maximum(av * inv128, 0.0)
        ar = jnp.concatenate([r0_ref[...], r1_ref[...]], axis=1)
        zr = jnp.maximum(ar * inv256, 0.0)
        out = (jnp.dot(zv, ov_ref[...], preferred_element_type=_f32)
               + jnp.dot(zr, or_ref[...], preferred_element_type=_f32)
               + ob_ref[...][0:1, :])
        a = sa_ref[0, 0]
        o_ref[...] = out * a + e_ref[...] * (1.0 - a)

    BN = N // 10
    return pl.pallas_call(
        body,
        grid=(10,),
        in_specs=[
            pl.BlockSpec((BN, 128), lambda i: (i, 0)),
            pl.BlockSpec((BN, 128), lambda i: (i, 0)),
            pl.BlockSpec((BN, 128), lambda i: (i, 0)),
            pl.BlockSpec((BN, 128), lambda i: (i, 0)),
            pl.BlockSpec((BN, 128), lambda i: (i, 0)),
            pl.BlockSpec((BN, 128), lambda i: (i, 0)),
            pl.BlockSpec((BN, 128), lambda i: (i, 0)),
            pl.BlockSpec((128, 128), lambda i: (0, 0)),
            pl.BlockSpec((256, 128), lambda i: (0, 0)),
            pl.BlockSpec((8, 128), lambda i: (0, 0)),
            pl.BlockSpec((8, 128), lambda i: (0, 0)),
            pl.BlockSpec((8, 256), lambda i: (0, 0)),
            pl.BlockSpec(memory_space=pltpu.SMEM),
        ],
        out_specs=pl.BlockSpec((BN, 128), lambda i: (i, 0)),
        out_shape=jax.ShapeDtypeStruct((N, 128), _f32),
    )(ar0, ar1, den0, den1, av0, av1, ent, ov, orr, outb8, exp4, exp8, a2)


_MESH = plsc.VectorSubcoreMesh(core_axis_name="c", subcore_axis_name="s")
_SC_PARAMS = pltpu.CompilerParams(needs_layout_passes=False,
                                  use_tc_tiling_on_sc=False)
_INV_SQRT = 1.0 / (DK ** 0.5)


@functools.partial(
    pl.kernel,
    out_type=[jax.ShapeDtypeStruct((E * H,), _f32),
              jax.ShapeDtypeStruct((NW * 16,), _f32)],
    mesh=_MESH,
    compiler_params=_SC_PARAMS,
    scratch_types=[
        pltpu.VMEM((CB,), _i32),
        pltpu.VMEM((CB,), _i32),
        pltpu.VMEM((CB, H), _f32),
        pltpu.VMEM((CB, H), _f32),
        pltpu.VMEM((CB * H,), _f32),
        pltpu.VMEM((16,), _f32),
        pltpu.SemaphoreType.DMA,
        pltpu.SemaphoreType.DMA,
    ],
)
def _sweep1(srcidx, dstidx, s_tab, t_tab, att_out, tmax_out,
            ia, ib, sbuf, tbuf, attbuf, maxbuf, sem_a, sem_b):
    wid = lax.axis_index("s") * NC + lax.axis_index("c")
    nblk = _nblk_of(wid)
    io16 = lax.iota(_i32, 16)

    def blk(i, m):
        b = wid + NW * i
        ca = pltpu.async_copy(srcidx.at[pl.ds(b * CB, CB)], ia, sem_a)
        cb = pltpu.async_copy(dstidx.at[pl.ds(b * CB, CB)], ib, sem_b)
        ca.wait()
        cb.wait()
        cs = pltpu.async_copy(s_tab.at[ia], sbuf, sem_a)
        ct = pltpu.async_copy(t_tab.at[ib], tbuf, sem_b)
        cs.wait()
        ct.wait()

        def vec(j, mm):
            fl = j * 16 + io16
            r = fl >> 2
            cc = fl & 3
            sv = plsc.load_gather(sbuf, [r, cc])
            tv = plsc.load_gather(tbuf, [r, cc])
            x = (sv + tv) * _INV_SQRT
            att = -jnp.where(x >= 0, x, ALPHA * x)
            attbuf[pl.ds(j * 16, 16)] = att
            return jnp.maximum(mm, att)

        m = pl.loop(0, CB * H // 16, init_carry=m)(vec)
        pltpu.sync_copy(attbuf, att_out.at[pl.ds(b * CB * H, CB * H)])
        return m

    m = pl.loop(0, nblk, init_carry=jnp.full((16,), -3.4e38, _f32))(blk)
    maxbuf[...] = m
    m = jnp.maximum(m, plsc.load_gather(maxbuf, [(io16 + 8) & 15]))
    maxbuf[...] = m
    m = jnp.maximum(m, plsc.load_gather(maxbuf, [(io16 + 4) & 15]))
    maxbuf[...] = m
    pltpu.sync_copy(maxbuf, tmax_out.at[pl.ds(wid * 16, 16)])


@functools.partial(
    pl.kernel,
    out_type=[jax.ShapeDtypeStruct((E * H,), _f32),
              jax.ShapeDtypeStruct((NC, N, 128), _f32)],
    mesh=_MESH,
    compiler_params=_SC_PARAMS,
    scratch_types=[
        pltpu.VMEM((NW * 16,), _f32),
        pltpu.VMEM((CB * H,), _f32),
        pltpu.VMEM((CB * H,), _f32),
        pltpu.VMEM((CB,), _i32),
        pltpu.VMEM((CB,), _i32),
        pltpu.VMEM((CB, 128), _f32),
        pltpu.VMEM_SHARED((N, 128), _f32),
        pltpu.SemaphoreType.DMA,
        pltpu.SemaphoreType.DMA,
        pltpu.SemaphoreType.DMA,
        pltpu.SemaphoreType.DMA,
    ],
)
def _sweep2(att_in, tmax_in, src, dst, v_tab, e_out, av_out,
            tmaxv, attbuf, ebuf, sidx, didx, vrows, av_sh,
            sem_a, sem_s, sem_d, sem_g):
    c = lax.axis_index("c")
    s = lax.axis_index("s")
    wid = s * NC + c
    nblk = _nblk_of(wid)

    pltpu.sync_copy(tmax_in, tmaxv)
    cvec = jnp.full((16,), -3.4e38, _f32)
    for j in range(NW):
        cvec = jnp.maximum(cvec, tmaxv[pl.ds(j * 16, 16)])

    def zr(j):
        vrows[j >> 3, pl.ds((j & 7) * 16, 16)] = jnp.zeros((16,), _f32)
    pl.loop(0, CB * 8)(zr)
    base = s * (N // NS)
    for tt in range(4):
        pltpu.sync_copy(vrows, av_sh.at[pl.ds(base + tt * CB, CB)])
    pltpu.sync_copy(vrows.at[pl.ds(0, N // NS - 4 * CB)],
                    av_sh.at[pl.ds(base + 4 * CB, N // NS - 4 * CB)])
    plsc.subcore_barrier()

    def blk(i, _):
        b = wid + NW * i
        ca = pltpu.async_copy(att_in.at[pl.ds(b * CB * H, CB * H)], attbuf,
                              sem_a)
        cs = pltpu.async_copy(src.at[pl.ds(b * CB, CB)], sidx, sem_s)
        cd = pltpu.async_copy(dst.at[pl.ds(b * CB, CB)], didx, sem_d)
        cs.wait()
        cg = pltpu.async_copy(v_tab.at[sidx], vrows, sem_g)
        ca.wait()
        for j in range(CB * H // 16):
            ebuf[pl.ds(j * 16, 16)] = jnp.exp(attbuf[pl.ds(j * 16, 16)] - cvec)
        pltpu.sync_copy(ebuf, e_out.at[pl.ds(b * CB * H, CB * H)])
        cg.wait()

        def scale(j):
            r = j >> 3
            cc = j & 7
            ei = r * H + (cc >> 1)
            ev = plsc.load_gather(ebuf, [jnp.broadcast_to(ei, (16,))])
            vrows[r, pl.ds(cc * 16, 16)] = vrows[r, pl.ds(cc * 16, 16)] * ev
        pl.loop(0, CB * 8)(scale)
        cd.wait()
        pltpu.sync_copy(vrows, av_sh.at[didx], add=True)
        return 0

    pl.loop(0, nblk, init_carry=0)(blk)
    plsc.subcore_barrier()
    for tt in range(4):
        pltpu.sync_copy(av_sh.at[pl.ds(base + tt * CB, CB)],
                        av_out.at[c, pl.ds(base + tt * CB, CB)])
    pltpu.sync_copy(av_sh.at[pl.ds(base + 4 * CB, N // NS - 4 * CB)],
                    av_out.at[c, pl.ds(base + 4 * CB, N // NS - 4 * CB)])


@functools.partial(
    pl.kernel,
    out_type=jax.ShapeDtypeStruct((NC, N, 128), _f32),
    mesh=_MESH,
    compiler_params=_SC_PARAMS,
    scratch_types=[
        pltpu.VMEM((CB * H,), _f32),
        pltpu.VMEM((CB,), _i32),
        pltpu.VMEM((CB,), _i32),
        pltpu.VMEM((CB,), _i32),
        pltpu.VMEM((CB, 128), _f32),
        pltpu.VMEM_SHARED((N, 128), _f32),
        pltpu.SemaphoreType.DMA,
        pltpu.SemaphoreType.DMA,
        pltpu.SemaphoreType.DMA,
        pltpu.SemaphoreType.DMA,
    ],
)
def _sweep3(e_in, et_in, dst, tt_tab, ar_out, ebuf, etb, gidx, didx, rrows,
            ar_sh, sem_a, sem_b, sem_d, sem_g):
    c = lax.axis_index("c")
    s = lax.axis_index("s")
    nblk = jnp.where(s < NBLK % NS, NBLK // NS + 1, NBLK // NS)

    def zr(j):
        rrows[j >> 3, pl.ds((j & 7) * 16, 16)] = jnp.zeros((16,), _f32)
    pl.loop(0, CB * 8)(zr)
    base = s * (N // NS)
    for tt in range(4):
        pltpu.sync_copy(rrows, ar_sh.at[pl.ds(base + tt * CB, CB)])
    pltpu.sync_copy(rrows.at[pl.ds(0, N // NS - 4 * CB)],
                    ar_sh.at[pl.ds(base + 4 * CB, N // NS - 4 * CB)])
    plsc.subcore_barrier()

    def blk(i, _):
        b = s + NS * i
        ca = pltpu.async_copy(e_in.at[pl.ds(b * CB * H, CB * H)], ebuf, sem_a)
        cb = pltpu.async_copy(et_in.at[pl.ds(b * CB, CB)], etb, sem_b)
        cd = pltpu.async_copy(dst.at[pl.ds(b * CB, CB)], didx, sem_d)
        cb.wait()
        for j in range(CB // 16):
            gidx[pl.ds(j * 16, 16)] = etb[pl.ds(j * 16, 16)] + c * R
        cg = pltpu.async_copy(tt_tab.at[gidx], rrows, sem_g)
        ca.wait()
        cg.wait()

        def scale(j):
            r = j >> 3
            cc = j & 7
            ei = r * H + c * 2 + (cc >> 2)
            ev = plsc.load_gather(ebuf, [jnp.broadcast_to(ei, (16,))])
            rrows[r, pl.ds(cc * 16, 16)] = rrows[r, pl.ds(cc * 16, 16)] * ev
        pl.loop(0, CB * 8)(scale)
        cd.wait()
        pltpu.sync_copy(rrows, ar_sh.at[didx], add=True)
        return 0

    pl.loop(0, nblk, init_carry=0)(blk)
    plsc.subcore_barrier()
    for tt in range(4):
        pltpu.sync_copy(ar_sh.at[pl.ds(base + tt * CB, CB)],
                        ar_out.at[c, pl.ds(base + tt * CB, CB)])
    pltpu.sync_copy(ar_sh.at[pl.ds(base + 4 * CB, N // NS - 4 * CB)],
                    ar_out.at[c, pl.ds(base + 4 * CB, N // NS - 4 * CB)])


@functools.partial(
    pl.kernel,
    out_type=jax.ShapeDtypeStruct((NC, N, 128), _f32),
    mesh=_MESH,
    compiler_params=_SC_PARAMS,
    scratch_types=[
        pltpu.VMEM((CB * H,), _f32),
        pltpu.VMEM((CB,), _i32),
        pltpu.VMEM((CB, 128), _f32),
        pltpu.VMEM_SHARED((N, 128), _f32),
        pltpu.SemaphoreType.DMA,
        pltpu.SemaphoreType.DMA,
        pltpu.SemaphoreType.DMA,
    ],
)
def _sweep4(e_in, dst, den_out, ebuf, didx, srows, den_sh,
            sem_a, sem_d, sem_g):
    c = lax.axis_index("c")
    s = lax.axis_index("s")
    wid = s * NC + c
    nblk = _nblk_of(wid)
    io16 = lax.iota(_i32, 16)

    def zr(j):
        srows[j >> 3, pl.ds((j & 7) * 16, 16)] = jnp.zeros((16,), _f32)
    pl.loop(0, CB * 8)(zr)
    base = s * (N // NS)
    for tt in range(4):
        pltpu.sync_copy(srows, den_sh.at[pl.ds(base + tt * CB, CB)])
    pltpu.sync_copy(srows.at[pl.ds(0, N // NS - 4 * CB)],
                    den_sh.at[pl.ds(base + 4 * CB, N // NS - 4 * CB)])
    plsc.subcore_barrier()

    def blk(i, _):
        b = wid + NW * i
        ca = pltpu.async_copy(e_in.at[pl.ds(b * CB * H, CB * H)], ebuf, sem_a)
        cd = pltpu.async_copy(dst.at[pl.ds(b * CB, CB)], didx, sem_d)
        ca.wait()
        for j in range(CB * H // 16):
            fl = j * 16 + io16
            plsc.store_scatter(srows, [fl >> 2, fl & 3],
                               ebuf[pl.ds(j * 16, 16)])
        cd.wait()
        pltpu.sync_copy(srows, den_sh.at[didx], add=True)
        for j in range(CB * H // 16):
            fl = j * 16 + io16
            plsc.store_scatter(srows, [fl >> 2, fl & 3],
                               jnp.zeros((16,), _f32))
        return 0

    pl.loop(0, nblk, init_carry=0)(blk)
    plsc.subcore_barrier()
    for tt in range(4):
        pltpu.sync_copy(den_sh.at[pl.ds(base + tt * CB, CB)],
                        den_out.at[c, pl.ds(base + tt * CB, CB)])
    pltpu.sync_copy(den_sh.at[pl.ds(base + 4 * CB, N // NS - 4 * CB)],
                    den_out.at[c, pl.ds(base + 4 * CB, N // NS - 4 * CB)])


def kernel(ent_name, ent_embed_in, edge_index, edge_type_r, r_head, r_head_sum,
           r_tail, r_tail_sum, kW, kb, qW, qb, vW, vb, outW, outb, attr_r,
           relL, relR, rel_skip_w):
    src = edge_index[0].astype(_i32)
    dst = edge_index[1].astype(_i32)
    et = edge_type_r.astype(_i32)

    def t(x):
        return x.transpose(1, 0, 2).reshape(x.shape[1], H * DK)
    wcat = jnp.concatenate([t(kW), t(qW), t(vW), t(relL), t(relR)], axis=1)
    wcat = jnp.pad(wcat, ((0, INP - 300), (0, 0)))
    bcat = jnp.concatenate([kb.reshape(-1), qb.reshape(-1), vb.reshape(-1),
                            jnp.zeros((256,), _f32)]).reshape(1, 640)
    bcat8 = jnp.broadcast_to(bcat, (8, 640))
    ent_pad = jnp.pad(ent_name, ((0, 0), (0, INP - 300)))

    proj = _proj_mm(ent_pad, wcat, bcat8)
    rel = _rel_mm(jnp.pad(r_head, ((0, 0), (0, KP - N))),
                  jnp.pad(r_tail, ((0, 0), (0, KP - N))),
                  jnp.pad(proj[:, 384:640], ((0, KP - N), (0, 0))),
                  jnp.broadcast_to(r_head_sum, (R, 128)),
                  jnp.broadcast_to(r_tail_sum, (R, 128)))

    eye4 = jnp.eye(H, dtype=_f32)
    rel_a = rel[:, :128].reshape(R, H, DK) * attr_r[None, :, :DK, 0]
    rel_b = rel[:, 128:].reshape(R, H, DK) * attr_r[None, :, DK:, 0]
    ms = jnp.pad((rel_a.transpose(1, 2, 0)[:, :, :, None]
                  * eye4[:, None, None, :]).reshape(H * DK, R, H),
                 ((0, 0), (0, RP - R), (0, 0))).reshape(H * DK, RP * H)
    mt = jnp.pad((rel_b.transpose(1, 2, 0)[:, :, :, None]
                  * eye4[:, None, None, :]).reshape(H * DK, R, H),
                 ((0, 0), (0, RP - R), (0, 0))).reshape(H * DK, RP * H)
    s_mat, t_mat = _st_mm(proj[:, :256], ms, mt)

    si2, di2 = _eidx_mm(src.reshape(E // 128, 128),
                        dst.reshape(E // 128, 128),
                        et.reshape(E // 128, 128))
    srcidx = si2.reshape(E)
    dstidx = di2.reshape(E)

    att_flat, tmax = _sweep1(srcidx, dstidx,
                             s_mat.reshape(N * RP, H), t_mat.reshape(N * RP, H))
    e_flat, av2 = _sweep2(att_flat, tmax, src, dst, proj[:, 256:384])

    relfull = jnp.concatenate([rel[:, :128].reshape(R, H, DK),
                               rel[:, 128:].reshape(R, H, DK)], axis=-1)
    tt_tab = relfull.reshape(R, 2, 128).transpose(1, 0, 2).reshape(2 * R, 128)
    ar2 = _sweep3(e_flat, et, dst, tt_tab)
    den2 = _sweep4(e_flat, dst)

    ov = (outW[:, :DK, :][:, :, None, :]
          * eye4[:, None, :, None]).reshape(H * DK, H * DK)
    orr = (outW[:, DK:, :][:, :, None, :]
           * eye4[:, None, :, None]).reshape(H * 2 * DK, H * DK)
    exp4 = jnp.pad(jnp.repeat(eye4, DK, axis=1), ((0, 4), (0, 0)))
    exp8 = jnp.pad(jnp.repeat(eye4, 2 * DK, axis=1), ((0, 4), (0, 0)))
    outb8 = jnp.broadcast_to(outb.reshape(1, H * DK), (8, H * DK))
    a2 = jax.nn.sigmoid(rel_skip_w).reshape(1, 1)

    return _final_mm(ar2[0], ar2[1], den2[0], den2[1], av2[0], av2[1],
                     ent_embed_in, ov, orr, outb8, exp4, exp8, a2)
